# baseline (speedup 1.0000x reference)
import functools

import jax
import jax.numpy as jnp
from jax import lax
from jax.experimental import pallas as pl
from jax.experimental.pallas import tpu as pltpu
from jax.experimental.pallas import tpu_sc as plsc

NC = 2
NS = 16
NW = NC * NS

N_NODES = 100000
MEM_DIM = 500
B = 16384
BPW = B // NW
GCH = 64
NCHUNK = BPW // GCH

_mesh = plsc.VectorSubcoreMesh(core_axis_name="c", subcore_axis_name="s",
                               num_cores=NC, num_subcores=NS)


def _gather_body(ids_hbm, mem_hbm, lu_hbm, h_out, lu_out,
                 idx_v, hbuf, lubuf, sem1, sem2):
    c = lax.axis_index("c")
    s = lax.axis_index("s")
    wid = s * NC + c
    base = wid * BPW
    pltpu.sync_copy(ids_hbm.at[pl.ds(base, BPW)], idx_v)
    for k in range(NCHUNK):
        row = idx_v.at[pl.ds(k * GCH, GCH)]
        cp2 = pltpu.async_copy(lu_hbm.at[row], lubuf, sem2)
        for q in range(GCH // 16):
            v = idx_v[pl.ds(k * GCH + q * 16, 16)]
            for j in range(16):
                pltpu.make_async_copy(
                    mem_hbm.at[pl.ds(v[j], 1)],
                    hbuf.at[pl.ds(q * 16 + j, 1)],
                    sem1,
                ).start()
        pltpu.make_async_copy(mem_hbm.at[pl.ds(0, GCH)], hbuf, sem1).wait()
        pltpu.sync_copy(hbuf, h_out.at[pl.ds(base + k * GCH, GCH)])
        cp2.wait()
        pltpu.sync_copy(lubuf, lu_out.at[pl.ds(base + k * GCH, GCH)])


_gather_call = pl.kernel(
    _gather_body,
    out_type=[
        jax.ShapeDtypeStruct((B, MEM_DIM), jnp.float32),
        jax.ShapeDtypeStruct((B,), jnp.float32),
    ],
    mesh=_mesh,
    scratch_types=[
        pltpu.VMEM((BPW,), jnp.int32),
        pltpu.VMEM((GCH, MEM_DIM), jnp.float32),
        pltpu.VMEM((GCH,), jnp.float32),
        pltpu.SemaphoreType.DMA,
        pltpu.SemaphoreType.DMA,
    ],
)


RW = 3128
RWL = N_NODES - 31 * RW
RWP = 3136
LOGB = 14


def _iota16():
    return jnp.arange(16, dtype=jnp.int32)


def _vgather16(x, sel):
    dnums = lax.GatherDimensionNumbers(
        offset_dims=(), collapsed_slice_dims=(0,), start_index_map=(0,))
    return lax.gather(x, sel[:, None], dnums, (1,),
                      mode=lax.GatherScatterMode.PROMISE_IN_BOUNDS)


def _scatter_body(ids_hbm, hnew_hbm, out_hbm,
                  ids_v, claim_v, wsrc_v, wdst_v, rowbuf, semg, sems):
    c = lax.axis_index("c")
    s = lax.axis_index("s")
    wid = s * NC + c
    lo = wid * RW
    hi = jnp.minimum(lo + RW, N_NODES)

    pltpu.sync_copy(ids_hbm, ids_v)

    def initbody(g, _):
        claim_v[pl.ds(g * 16, 16)] = jnp.full((16,), -1, jnp.int32)
        return _
    lax.fori_loop(0, RWP // 16, initbody, 0)

    iota = _iota16()
    shifted_sel = jnp.minimum(iota + 1, 15)

    def scanbody(t, _):
        n = ids_v[pl.ds(t * 16, 16)]
        key = (n << LOGB) | (t * 16 + iota)
        srt = lax.sort(key)
        n_s = srt >> LOGB
        i_s = srt & (B - 1)
        nxt = _vgather16(n_s, shifted_sel)
        win = (iota == 15) | (nxt != n_s)
        m = win & (n_s >= lo) & (n_s < hi)
        plsc.store_scatter(claim_v, [n_s - lo], i_s, mask=m)
        return _
    lax.fori_loop(0, B // 16, scanbody, 0)

    def compactbody(g, off):
        v = claim_v[pl.ds(g * 16, 16)]
        m = v >= 0
        plsc.store_compressed(wsrc_v.at[pl.ds(off, 16)], v, mask=m)
        plsc.store_compressed(wdst_v.at[pl.ds(off, 16)], lo + g * 16 + iota,
                              mask=m)
        return off + jnp.sum(jnp.where(m, 1, 0))
    cnt = lax.fori_loop(0, RWP // 16, compactbody, jnp.int32(0))

    def chunk(src16, dst16):
        for j in range(16):
            pltpu.make_async_copy(hnew_hbm.at[pl.ds(src16[j], 1)],
                                  rowbuf.at[pl.ds(j, 1)], semg).start()
        pltpu.make_async_copy(hnew_hbm.at[pl.ds(0, 16)], rowbuf, semg).wait()
        for j in range(16):
            pltpu.make_async_copy(rowbuf.at[pl.ds(j, 1)],
                                  out_hbm.at[pl.ds(dst16[j], 1)], sems).start()
        pltpu.make_async_copy(rowbuf, out_hbm.at[pl.ds(0, 16)], sems).wait()

    def scatbody(t, _):
        chunk(wsrc_v[pl.ds(t * 16, 16)], wdst_v[pl.ds(t * 16, 16)])
        return _
    lax.fori_loop(0, cnt >> 4, scatbody, 0)

    @pl.when(cnt > 0)
    def _():
        sel = jnp.maximum(iota + cnt - 16, 0)
        chunk(plsc.load_gather(wsrc_v, [sel]),
              plsc.load_gather(wdst_v, [sel]))


_scatter_call = pl.kernel(
    _scatter_body,
    out_type=(),
    mesh=_mesh,
    scratch_types=[
        pltpu.VMEM((B,), jnp.int32),
        pltpu.VMEM((RWP,), jnp.int32),
        pltpu.VMEM((RWP,), jnp.int32),
        pltpu.VMEM((RWP,), jnp.int32),
        pltpu.VMEM((16, MEM_DIM), jnp.float32),
        pltpu.SemaphoreType.DMA,
        pltpu.SemaphoreType.DMA,
    ],
    compiler_params=pltpu.CompilerParams(needs_layout_passes=False),
)


BB = 512
NBB = B // BB


CPB = 3136
CPL = N_NODES - 31 * CPB


def _tc_body(mem, h, ef, et, lu, wt, bt, w1a, w1b, w1c, b1, w2, b2,
             wxr, wxz, wxn, whr, whz, whn, bxr, bxz, bxn, bhr, bhz, bhn,
             hn_out, tab_out, csem):
    f32 = jnp.float32
    bf16 = jnp.bfloat16
    i = pl.program_id(0)

    @pl.when(i == 0)
    def _():
        for k in range(NBB - 1):
            pltpu.make_async_copy(mem.at[pl.ds(k * CPB, CPB)],
                                  tab_out.at[pl.ds(k * CPB, CPB)],
                                  csem).start()
        pltpu.make_async_copy(mem.at[pl.ds((NBB - 1) * CPB, CPL)],
                              tab_out.at[pl.ds((NBB - 1) * CPB, CPL)],
                              csem).start()

    @pl.when(i == NBB - 1)
    def _():
        for k in range(NBB - 1):
            pltpu.make_async_copy(mem.at[pl.ds(k * CPB, CPB)],
                                  tab_out.at[pl.ds(k * CPB, CPB)],
                                  csem).wait()
        pltpu.make_async_copy(mem.at[pl.ds((NBB - 1) * CPB, CPL)],
                              tab_out.at[pl.ds((NBB - 1) * CPB, CPL)],
                              csem).wait()

    hh = h[...]
    td = et[...] - lu[...]
    te = jnp.cos(td * wt[...] + bt[...])
    hb = hh.astype(bf16)

    def bdot(a, w):
        return jnp.dot(a.astype(bf16), w[...].astype(bf16),
                       preferred_element_type=f32)

    hid = (bdot(hh, w1a) + bdot(ef[...], w1b) + bdot(te, w1c) + b1[...])
    hid = jnp.maximum(hid, 0.0)
    msg = bdot(hid, w2) + b2[...]
    xr = bdot(msg, wxr) + bxr[...]
    xz = bdot(msg, wxz) + bxz[...]
    xn = bdot(msg, wxn) + bxn[...]
    hr = jnp.dot(hb, whr[...].astype(bf16), preferred_element_type=f32) + bhr[...]
    hz = jnp.dot(hb, whz[...].astype(bf16), preferred_element_type=f32) + bhz[...]
    hn = jnp.dot(hb, whn[...].astype(bf16), preferred_element_type=f32) + bhn[...]
    r = jax.nn.sigmoid(xr + hr)
    z = jax.nn.sigmoid(xz + hz)
    n = jnp.tanh(xn + r * hn)
    hn_out[...] = (1.0 - z) * n + z * hh


def _const_spec(shape):
    nd = len(shape)
    return pl.BlockSpec(shape, lambda i: (0,) * nd)


def _tc_compute(mem, h, ef, et1, lu1, wt2, bt2, w1a, w1b, w1c, b1_2, w2, b2_2,
                gw, gb):
    in_specs = [
        pl.BlockSpec(memory_space=pl.ANY),
        pl.BlockSpec((BB, MEM_DIM), lambda i: (i, 0)),
        pl.BlockSpec((BB, ef.shape[1]), lambda i: (i, 0)),
        pl.BlockSpec((BB, 1), lambda i: (i, 0)),
        pl.BlockSpec((BB, 1), lambda i: (i, 0)),
        _const_spec(wt2.shape),
        _const_spec(bt2.shape),
        _const_spec(w1a.shape),
        _const_spec(w1b.shape),
        _const_spec(w1c.shape),
        _const_spec(b1_2.shape),
        _const_spec(w2.shape),
        _const_spec(b2_2.shape),
    ] + [_const_spec(w.shape) for w in gw] + [_const_spec(b.shape) for b in gb]
    return pl.pallas_call(
        _tc_body,
        grid=(NBB,),
        in_specs=in_specs,
        out_specs=[
            pl.BlockSpec((BB, MEM_DIM), lambda i: (i, 0)),
            pl.BlockSpec(memory_space=pl.ANY),
        ],
        out_shape=[
            jax.ShapeDtypeStruct((B, MEM_DIM), jnp.float32),
            jax.ShapeDtypeStruct((N_NODES, MEM_DIM), jnp.float32),
        ],
        scratch_shapes=[pltpu.SemaphoreType.DMA],
    )(mem, h, ef, et1, lu1, wt2, bt2, w1a, w1b, w1c, b1_2, w2, b2_2, *gw, *gb)


def kernel(memory, last_update, node_ids, edge_feats, edge_times,
           w_t, b_t, W1, b1, W2, b2, W_ih, b_ih, W_hh, b_hh):
    ids = node_ids.astype(jnp.int32)
    h, lu = _gather_call(ids, memory, last_update)

    w1a = W1[:MEM_DIM]
    w1b = W1[MEM_DIM:MEM_DIM + 17]
    w1c = W1[MEM_DIM + 17:]
    gw = [W_ih[:, :MEM_DIM], W_ih[:, MEM_DIM:2 * MEM_DIM], W_ih[:, 2 * MEM_DIM:],
          W_hh[:, :MEM_DIM], W_hh[:, MEM_DIM:2 * MEM_DIM], W_hh[:, 2 * MEM_DIM:]]
    gb = [b_ih[:MEM_DIM][None], b_ih[MEM_DIM:2 * MEM_DIM][None], b_ih[2 * MEM_DIM:][None],
          b_hh[:MEM_DIM][None], b_hh[MEM_DIM:2 * MEM_DIM][None], b_hh[2 * MEM_DIM:][None]]
    h_new, table = _tc_compute(memory, h, edge_feats, edge_times[:, None],
                               lu[:, None], w_t[None], b_t[None],
                               w1a, w1b, w1c, b1[None], W2, b2[None], gw, gb)

    ref = jax.new_ref(table)
    _scatter_call(ids, h_new, ref)
    return jax.freeze(ref)


# speedup vs baseline: 9.0504x; 9.0504x over previous
import functools

import jax
import jax.numpy as jnp
from jax import lax
from jax.experimental import pallas as pl
from jax.experimental.pallas import tpu as pltpu
from jax.experimental.pallas import tpu_sc as plsc

NC = 2
NS = 16
NW = NC * NS

N_NODES = 100000
MEM_DIM = 500
B = 16384
BPW = B // NW
GCH = 64
NCHUNK = BPW // GCH

_mesh = plsc.VectorSubcoreMesh(core_axis_name="c", subcore_axis_name="s",
                               num_cores=NC, num_subcores=NS)


def _gather_body(ids_hbm, mem_hbm, lu_hbm, h_out, lu_out,
                 idx_v, hbuf, lubuf, sem1, sem2):
    c = lax.axis_index("c")
    s = lax.axis_index("s")
    wid = s * NC + c
    base = wid * BPW
    pltpu.sync_copy(ids_hbm.at[pl.ds(base, BPW)], idx_v)
    for k in range(NCHUNK):
        row = idx_v.at[pl.ds(k * GCH, GCH)]
        cp2 = pltpu.async_copy(lu_hbm.at[row], lubuf, sem2)
        for q in range(GCH // 16):
            v = idx_v[pl.ds(k * GCH + q * 16, 16)]
            for j in range(16):
                pltpu.make_async_copy(
                    mem_hbm.at[pl.ds(v[j], 1)],
                    hbuf.at[pl.ds(q * 16 + j, 1)],
                    sem1,
                ).start()
        pltpu.make_async_copy(mem_hbm.at[pl.ds(0, GCH)], hbuf, sem1).wait()
        pltpu.sync_copy(hbuf, h_out.at[pl.ds(base + k * GCH, GCH)])
        cp2.wait()
        pltpu.sync_copy(lubuf, lu_out.at[pl.ds(base + k * GCH, GCH)])


_gather_call = pl.kernel(
    _gather_body,
    out_type=[
        jax.ShapeDtypeStruct((B, MEM_DIM), jnp.float32),
        jax.ShapeDtypeStruct((B,), jnp.float32),
    ],
    mesh=_mesh,
    scratch_types=[
        pltpu.VMEM((BPW,), jnp.int32),
        pltpu.VMEM((GCH, MEM_DIM), jnp.float32),
        pltpu.VMEM((GCH,), jnp.float32),
        pltpu.SemaphoreType.DMA,
        pltpu.SemaphoreType.DMA,
    ],
)


RW = 3128
RWL = N_NODES - 31 * RW
RWP = 3136
LOGB = 14


def _iota16():
    return jnp.arange(16, dtype=jnp.int32)


def _vgather16(x, sel):
    dnums = lax.GatherDimensionNumbers(
        offset_dims=(), collapsed_slice_dims=(0,), start_index_map=(0,))
    return lax.gather(x, sel[:, None], dnums, (1,),
                      mode=lax.GatherScatterMode.PROMISE_IN_BOUNDS)


def _scatter_body(ids_hbm, hnew_hbm, out_hbm,
                  ids_v, claim_v, wsrc_v, wdst_v, rowbuf, semg, sems):
    c = lax.axis_index("c")
    s = lax.axis_index("s")
    wid = s * NC + c
    lo = wid * RW
    hi = jnp.minimum(lo + RW, N_NODES)

    pltpu.sync_copy(ids_hbm, ids_v)

    def initbody(g, _):
        claim_v[pl.ds(g * 16, 16)] = jnp.full((16,), -1, jnp.int32)
        return _
    lax.fori_loop(0, RWP // 16, initbody, 0)

    iota = _iota16()
    shifted_sel = jnp.minimum(iota + 1, 15)

    def scanbody(t, _):
        n = ids_v[pl.ds(t * 16, 16)]
        key = (n << LOGB) | (t * 16 + iota)
        srt = lax.sort(key)
        n_s = srt >> LOGB
        i_s = srt & (B - 1)
        nxt = _vgather16(n_s, shifted_sel)
        win = (iota == 15) | (nxt != n_s)
        m = win & (n_s >= lo) & (n_s < hi)
        plsc.store_scatter(claim_v, [n_s - lo], i_s, mask=m)
        return _
    lax.fori_loop(0, B // 16, scanbody, 0)

    def compactbody(g, off):
        v = claim_v[pl.ds(g * 16, 16)]
        m = v >= 0
        plsc.store_compressed(wsrc_v.at[pl.ds(off, 16)], v, mask=m)
        plsc.store_compressed(wdst_v.at[pl.ds(off, 16)], lo + g * 16 + iota,
                              mask=m)
        return off + jnp.sum(jnp.where(m, 1, 0))
    cnt = lax.fori_loop(0, RWP // 16, compactbody, jnp.int32(0))

    def chunk(src16, dst16):
        for j in range(16):
            pltpu.make_async_copy(hnew_hbm.at[pl.ds(src16[j], 1)],
                                  rowbuf.at[pl.ds(j, 1)], semg).start()
        pltpu.make_async_copy(hnew_hbm.at[pl.ds(0, 16)], rowbuf, semg).wait()
        for j in range(16):
            pltpu.make_async_copy(rowbuf.at[pl.ds(j, 1)],
                                  out_hbm.at[pl.ds(dst16[j], 1)], sems).start()
        pltpu.make_async_copy(rowbuf, out_hbm.at[pl.ds(0, 16)], sems).wait()

    def scatbody(t, _):
        chunk(wsrc_v[pl.ds(t * 16, 16)], wdst_v[pl.ds(t * 16, 16)])
        return _
    lax.fori_loop(0, cnt >> 4, scatbody, 0)

    @pl.when(cnt > 0)
    def _():
        sel = jnp.maximum(iota + cnt - 16, 0)
        chunk(plsc.load_gather(wsrc_v, [sel]),
              plsc.load_gather(wdst_v, [sel]))


_scatter_call = pl.kernel(
    _scatter_body,
    out_type=(),
    mesh=_mesh,
    scratch_types=[
        pltpu.VMEM((B,), jnp.int32),
        pltpu.VMEM((RWP,), jnp.int32),
        pltpu.VMEM((RWP,), jnp.int32),
        pltpu.VMEM((RWP,), jnp.int32),
        pltpu.VMEM((16, MEM_DIM), jnp.float32),
        pltpu.SemaphoreType.DMA,
        pltpu.SemaphoreType.DMA,
    ],
    compiler_params=pltpu.CompilerParams(needs_layout_passes=False),
)


BB = 512
NBB = B // BB


CPB = 3136
CPL = N_NODES - 31 * CPB


def _copy_body(mem, tab_out):
    tab_out[...] = mem[...]


def _tc_body(h, ef, et, lu, wt, bt, w1a, w1b, w1c, b1, w2, b2,
             wxr, wxz, wxn, whr, whz, whn, bxr, bxz, bxn, bhr, bhz, bhn,
             hn_out):
    f32 = jnp.float32
    bf16 = jnp.bfloat16
    hh = h[...]
    td = et[...] - lu[...]
    te = jnp.cos(td * wt[...] + bt[...])
    hb = hh.astype(bf16)

    def bdot(a, w):
        return jnp.dot(a.astype(bf16), w[...].astype(bf16),
                       preferred_element_type=f32)

    hid = (bdot(hh, w1a) + bdot(ef[...], w1b) + bdot(te, w1c) + b1[...])
    hid = jnp.maximum(hid, 0.0)
    msg = bdot(hid, w2) + b2[...]
    xr = bdot(msg, wxr) + bxr[...]
    xz = bdot(msg, wxz) + bxz[...]
    xn = bdot(msg, wxn) + bxn[...]
    hr = jnp.dot(hb, whr[...].astype(bf16), preferred_element_type=f32) + bhr[...]
    hz = jnp.dot(hb, whz[...].astype(bf16), preferred_element_type=f32) + bhz[...]
    hn = jnp.dot(hb, whn[...].astype(bf16), preferred_element_type=f32) + bhn[...]
    r = jax.nn.sigmoid(xr + hr)
    z = jax.nn.sigmoid(xz + hz)
    n = jnp.tanh(xn + r * hn)
    hn_out[...] = (1.0 - z) * n + z * hh


def _const_spec(shape):
    nd = len(shape)
    return pl.BlockSpec(shape, lambda i: (0,) * nd)


def _table_copy(mem):
    return pl.pallas_call(
        _copy_body,
        grid=(NBB,),
        in_specs=[pl.BlockSpec((CPB, MEM_DIM), lambda i: (i, 0))],
        out_specs=pl.BlockSpec((CPB, MEM_DIM), lambda i: (i, 0)),
        out_shape=jax.ShapeDtypeStruct((N_NODES, MEM_DIM), jnp.float32),
    )(mem)


def _tc_compute(h, ef, et1, lu1, wt2, bt2, w1a, w1b, w1c, b1_2, w2, b2_2,
                gw, gb):
    in_specs = [
        pl.BlockSpec((BB, MEM_DIM), lambda i: (i, 0)),
        pl.BlockSpec((BB, ef.shape[1]), lambda i: (i, 0)),
        pl.BlockSpec((BB, 1), lambda i: (i, 0)),
        pl.BlockSpec((BB, 1), lambda i: (i, 0)),
        _const_spec(wt2.shape),
        _const_spec(bt2.shape),
        _const_spec(w1a.shape),
        _const_spec(w1b.shape),
        _const_spec(w1c.shape),
        _const_spec(b1_2.shape),
        _const_spec(w2.shape),
        _const_spec(b2_2.shape),
    ] + [_const_spec(w.shape) for w in gw] + [_const_spec(b.shape) for b in gb]
    return pl.pallas_call(
        _tc_body,
        grid=(NBB,),
        in_specs=in_specs,
        out_specs=pl.BlockSpec((BB, MEM_DIM), lambda i: (i, 0)),
        out_shape=jax.ShapeDtypeStruct((B, MEM_DIM), jnp.float32),
    )(h, ef, et1, lu1, wt2, bt2, w1a, w1b, w1c, b1_2, w2, b2_2, *gw, *gb)


def kernel(memory, last_update, node_ids, edge_feats, edge_times,
           w_t, b_t, W1, b1, W2, b2, W_ih, b_ih, W_hh, b_hh):
    ids = node_ids.astype(jnp.int32)
    h, lu = _gather_call(ids, memory, last_update)

    w1a = W1[:MEM_DIM]
    w1b = W1[MEM_DIM:MEM_DIM + 17]
    w1c = W1[MEM_DIM + 17:]
    gw = [W_ih[:, :MEM_DIM], W_ih[:, MEM_DIM:2 * MEM_DIM], W_ih[:, 2 * MEM_DIM:],
          W_hh[:, :MEM_DIM], W_hh[:, MEM_DIM:2 * MEM_DIM], W_hh[:, 2 * MEM_DIM:]]
    gb = [b_ih[:MEM_DIM][None], b_ih[MEM_DIM:2 * MEM_DIM][None], b_ih[2 * MEM_DIM:][None],
          b_hh[:MEM_DIM][None], b_hh[MEM_DIM:2 * MEM_DIM][None], b_hh[2 * MEM_DIM:][None]]
    table = _table_copy(memory)
    h_new = _tc_compute(h, edge_feats, edge_times[:, None],
                        lu[:, None], w_t[None], b_t[None],
                        w1a, w1b, w1c, b1[None], W2, b2[None], gw, gb)

    ref = jax.new_ref(table)
    _scatter_call(ids, h_new, ref)
    return jax.freeze(ref)


# speedup vs baseline: 11.0316x; 1.2189x over previous
import functools

import jax
import jax.numpy as jnp
from jax import lax
from jax.experimental import pallas as pl
from jax.experimental.pallas import tpu as pltpu
from jax.experimental.pallas import tpu_sc as plsc

NC = 2
NS = 16
NW = NC * NS

N_NODES = 100000
MEM_DIM = 500
B = 16384
BPW = B // NW
GCH = 64
NCHUNK = BPW // GCH

_mesh = plsc.VectorSubcoreMesh(core_axis_name="c", subcore_axis_name="s",
                               num_cores=NC, num_subcores=NS)


def _gather_body(ids_hbm, mem_hbm, lu_hbm, h_out, lu_out,
                 idx_v, hbuf, lubuf, sem1, sem2):
    c = lax.axis_index("c")
    s = lax.axis_index("s")
    wid = s * NC + c
    base = wid * BPW
    pltpu.sync_copy(ids_hbm.at[pl.ds(base, BPW)], idx_v)
    for k in range(NCHUNK):
        row = idx_v.at[pl.ds(k * GCH, GCH)]
        cp2 = pltpu.async_copy(lu_hbm.at[row], lubuf, sem2)
        for q in range(GCH // 16):
            v = idx_v[pl.ds(k * GCH + q * 16, 16)]
            for j in range(16):
                pltpu.make_async_copy(
                    mem_hbm.at[pl.ds(v[j], 1)],
                    hbuf.at[pl.ds(q * 16 + j, 1)],
                    sem1,
                ).start()
        pltpu.make_async_copy(mem_hbm.at[pl.ds(0, GCH)], hbuf, sem1).wait()
        pltpu.sync_copy(hbuf, h_out.at[pl.ds(base + k * GCH, GCH)])
        cp2.wait()
        pltpu.sync_copy(lubuf, lu_out.at[pl.ds(base + k * GCH, GCH)])


_gather_call = pl.kernel(
    _gather_body,
    out_type=[
        jax.ShapeDtypeStruct((B, MEM_DIM), jnp.float32),
        jax.ShapeDtypeStruct((B,), jnp.float32),
    ],
    mesh=_mesh,
    scratch_types=[
        pltpu.VMEM((BPW,), jnp.int32),
        pltpu.VMEM((GCH, MEM_DIM), jnp.float32),
        pltpu.VMEM((GCH,), jnp.float32),
        pltpu.SemaphoreType.DMA,
        pltpu.SemaphoreType.DMA,
    ],
)


RW = 3128
RWL = N_NODES - 31 * RW
RWP = 3136
LOGB = 14


def _iota16():
    return jnp.arange(16, dtype=jnp.int32)


def _vgather16(x, sel):
    dnums = lax.GatherDimensionNumbers(
        offset_dims=(), collapsed_slice_dims=(0,), start_index_map=(0,))
    return lax.gather(x, sel[:, None], dnums, (1,),
                      mode=lax.GatherScatterMode.PROMISE_IN_BOUNDS)


def _scatter_body(ids_hbm, hnew_hbm, out_hbm,
                  ids_v, claim_v, wsrc_v, wdst_v, rowbuf, semg, sems):
    c = lax.axis_index("c")
    s = lax.axis_index("s")
    wid = s * NC + c
    lo = wid * RW
    hi = jnp.minimum(lo + RW, N_NODES)

    pltpu.sync_copy(ids_hbm, ids_v)

    def initbody(g, _):
        claim_v[pl.ds(g * 16, 16)] = jnp.full((16,), -1, jnp.int32)
        return _
    lax.fori_loop(0, RWP // 16, initbody, 0)

    iota = _iota16()
    shifted_sel = jnp.minimum(iota + 1, 15)

    def scanbody(t, _):
        n = ids_v[pl.ds(t * 16, 16)]
        key = (n << LOGB) | (t * 16 + iota)
        srt = lax.sort(key)
        n_s = srt >> LOGB
        i_s = srt & (B - 1)
        nxt = _vgather16(n_s, shifted_sel)
        win = (iota == 15) | (nxt != n_s)
        m = win & (n_s >= lo) & (n_s < hi)
        plsc.store_scatter(claim_v, [n_s - lo], i_s, mask=m)
        return _
    lax.fori_loop(0, B // 16, scanbody, 0)

    def compactbody(g, off):
        v = claim_v[pl.ds(g * 16, 16)]
        m = v >= 0
        plsc.store_compressed(wsrc_v.at[pl.ds(off, 16)], v, mask=m)
        plsc.store_compressed(wdst_v.at[pl.ds(off, 16)], lo + g * 16 + iota,
                              mask=m)
        return off + jnp.sum(jnp.where(m, 1, 0))
    cnt = lax.fori_loop(0, RWP // 16, compactbody, jnp.int32(0))

    def chunk(src16, dst16):
        for j in range(16):
            pltpu.make_async_copy(hnew_hbm.at[pl.ds(src16[j], 1)],
                                  rowbuf.at[pl.ds(j, 1)], semg).start()
        pltpu.make_async_copy(hnew_hbm.at[pl.ds(0, 16)], rowbuf, semg).wait()
        for j in range(16):
            pltpu.make_async_copy(rowbuf.at[pl.ds(j, 1)],
                                  out_hbm.at[pl.ds(dst16[j], 1)], sems).start()
        pltpu.make_async_copy(rowbuf, out_hbm.at[pl.ds(0, 16)], sems).wait()

    def scatbody(t, _):
        chunk(wsrc_v[pl.ds(t * 16, 16)], wdst_v[pl.ds(t * 16, 16)])
        return _
    lax.fori_loop(0, cnt >> 4, scatbody, 0)

    @pl.when(cnt > 0)
    def _():
        sel = jnp.maximum(iota + cnt - 16, 0)
        chunk(plsc.load_gather(wsrc_v, [sel]),
              plsc.load_gather(wdst_v, [sel]))


_scatter_call = pl.kernel(
    _scatter_body,
    out_type=(),
    mesh=_mesh,
    scratch_types=[
        pltpu.VMEM((B,), jnp.int32),
        pltpu.VMEM((RWP,), jnp.int32),
        pltpu.VMEM((RWP,), jnp.int32),
        pltpu.VMEM((RWP,), jnp.int32),
        pltpu.VMEM((16, MEM_DIM), jnp.float32),
        pltpu.SemaphoreType.DMA,
        pltpu.SemaphoreType.DMA,
    ],
    compiler_params=pltpu.CompilerParams(needs_layout_passes=False),
)


BB = 512
NBB = B // BB


CPB = 3136
CPL = N_NODES - 31 * CPB


def _copy_body(mem, tab_out):
    tab_out[...] = mem[...]


def _tc_body(h, ef, et, lu, wt, bt, w1a, w1b, w1c, b1, w2, b2,
             wxr, wxz, wxn, whr, whz, whn, bxr, bxz, bxn, bhr, bhz, bhn,
             hn_out):
    f32 = jnp.float32
    bf16 = jnp.bfloat16
    hh = h[...]
    td = et[...] - lu[...]
    te = jnp.cos(td * wt[...] + bt[...])
    hb = hh.astype(bf16)

    def bdot(a, w):
        return jnp.dot(a.astype(bf16), w[...].astype(bf16),
                       preferred_element_type=f32)

    hid = (bdot(hh, w1a) + bdot(ef[...], w1b) + bdot(te, w1c) + b1[...])
    hid = jnp.maximum(hid, 0.0)
    msg = bdot(hid, w2) + b2[...]
    xr = bdot(msg, wxr) + bxr[...]
    xz = bdot(msg, wxz) + bxz[...]
    xn = bdot(msg, wxn) + bxn[...]
    hr = jnp.dot(hb, whr[...].astype(bf16), preferred_element_type=f32) + bhr[...]
    hz = jnp.dot(hb, whz[...].astype(bf16), preferred_element_type=f32) + bhz[...]
    hn = jnp.dot(hb, whn[...].astype(bf16), preferred_element_type=f32) + bhn[...]
    r = jax.nn.sigmoid(xr + hr)
    z = jax.nn.sigmoid(xz + hz)
    n = jnp.tanh(xn + r * hn)
    hn_out[...] = (1.0 - z) * n + z * hh


def _const_spec(shape):
    nd = len(shape)
    return pl.BlockSpec(shape, lambda i: (0,) * nd)


def _table_copy(mem):
    return pl.pallas_call(
        _copy_body,
        grid=(NBB,),
        in_specs=[pl.BlockSpec((CPB, MEM_DIM), lambda i: (i, 0))],
        out_specs=pl.BlockSpec((CPB, MEM_DIM), lambda i: (i, 0)),
        out_shape=jax.ShapeDtypeStruct((N_NODES, MEM_DIM), jnp.float32),
    )(mem)


def _tc_compute(h, ef, et1, lu1, wt2, bt2, w1a, w1b, w1c, b1_2, w2, b2_2,
                gw, gb):
    in_specs = [
        pl.BlockSpec((BB, MEM_DIM), lambda i: (i, 0)),
        pl.BlockSpec((BB, ef.shape[1]), lambda i: (i, 0)),
        pl.BlockSpec((BB, 1), lambda i: (i, 0)),
        pl.BlockSpec((BB, 1), lambda i: (i, 0)),
        _const_spec(wt2.shape),
        _const_spec(bt2.shape),
        _const_spec(w1a.shape),
        _const_spec(w1b.shape),
        _const_spec(w1c.shape),
        _const_spec(b1_2.shape),
        _const_spec(w2.shape),
        _const_spec(b2_2.shape),
    ] + [_const_spec(w.shape) for w in gw] + [_const_spec(b.shape) for b in gb]
    return pl.pallas_call(
        _tc_body,
        grid=(NBB,),
        in_specs=in_specs,
        out_specs=pl.BlockSpec((BB, MEM_DIM), lambda i: (i, 0)),
        out_shape=jax.ShapeDtypeStruct((B, MEM_DIM), jnp.float32),
    )(h, ef, et1, lu1, wt2, bt2, w1a, w1b, w1c, b1_2, w2, b2_2, *gw, *gb)


def kernel(memory, last_update, node_ids, edge_feats, edge_times,
           w_t, b_t, W1, b1, W2, b2, W_ih, b_ih, W_hh, b_hh):
    ids = node_ids.astype(jnp.int32)
    h, lu = _gather_call(ids, memory, last_update)

    w1a = W1[:MEM_DIM]
    w1b = W1[MEM_DIM:MEM_DIM + 17]
    w1c = W1[MEM_DIM + 17:]
    gw = [W_ih[:, :MEM_DIM], W_ih[:, MEM_DIM:2 * MEM_DIM], W_ih[:, 2 * MEM_DIM:],
          W_hh[:, :MEM_DIM], W_hh[:, MEM_DIM:2 * MEM_DIM], W_hh[:, 2 * MEM_DIM:]]
    gb = [b_ih[:MEM_DIM][None], b_ih[MEM_DIM:2 * MEM_DIM][None], b_ih[2 * MEM_DIM:][None],
          b_hh[:MEM_DIM][None], b_hh[MEM_DIM:2 * MEM_DIM][None], b_hh[2 * MEM_DIM:][None]]
    h_new = _tc_compute(h, edge_feats, edge_times[:, None],
                        lu[:, None], w_t[None], b_t[None],
                        w1a, w1b, w1c, b1[None], W2, b2[None], gw, gb)

    ref = jax.new_ref(memory)
    _scatter_call(ids, h_new, ref)
    return jax.freeze(ref)
